# Initial kernel scaffold; baseline (speedup 1.0000x reference)
import jax
import jax.numpy as jnp
from jax.experimental import pallas as pl


def kernel(feature_embedding, field_idx, field_sub_idx, feature_idx, feature_vals, batch_idx):
    raise NotImplementedError("write your pallas kernel here")


# speedup vs baseline: 1.8452x; 1.8452x over previous
import functools

import jax
import jax.numpy as jnp
from jax import lax
from jax.experimental import pallas as pl
from jax.experimental.pallas import tpu as pltpu
from jax.experimental.pallas import tpu_sc as plsc

BATCH = 4096
FIELD_DIM = 26
VOCAB = 100000
EMBED = 64
NNZ = BATCH * FIELD_DIM

NCORE = 2
NSUB = 16
CPC = 2
NCHUNK = NCORE * CPC
CHUNK_B = BATCH // NCHUNK
CROWS = CHUNK_B * FIELD_DIM
TROWS = CROWS // NSUB
K = 128
ZROWS = 104
PAD = 16 * K + 8


def _mesh_kernel():
    mesh = plsc.VectorSubcoreMesh(core_axis_name="c", subcore_axis_name="s")

    @functools.partial(
        pl.kernel,
        mesh=mesh,
        out_type=jax.ShapeDtypeStruct((NNZ, EMBED), jnp.float32),
        compiler_params=pltpu.CompilerParams(use_tc_tiling_on_sc=False),
        scratch_types=[
            pltpu.VMEM((16,), jnp.int32),
            pltpu.VMEM((K,), jnp.int32),
            pltpu.VMEM((K,), jnp.int32),
            pltpu.VMEM((K,), jnp.int32),
            pltpu.VMEM((K,), jnp.float32),
            pltpu.VMEM((K,), jnp.float32),
            pltpu.VMEM((K,), jnp.int32),
            pltpu.VMEM((K, EMBED), jnp.float32),
            pltpu.VMEM((ZROWS, EMBED), jnp.float32),
            pltpu.VMEM_SHARED((CROWS, EMBED), jnp.float32),
            pltpu.SemaphoreType.DMA,
        ],
    )
    def k(meta_hbm, bi_hbm, fi_hbm, fidx_hbm, val_hbm, table_hbm, out_hbm,
          meta_v, bi_v, fi_v, fidx_v, vraw_v, val_v, lrow_v, rows_v,
          zbuf, acc_sh, sem):
        cid = lax.axis_index("c")
        sid = lax.axis_index("s")
        wid = sid * NCORE + cid
        lane = lax.broadcasted_iota(jnp.int32, (16,), 0)
        z16 = jnp.zeros((16,), jnp.float32)

        def zrow(i, _):
            for c in range(EMBED // 16):
                zbuf[i, pl.ds(c * 16, 16)] = z16
            return 0

        lax.fori_loop(0, ZROWS, zrow, 0)

        pltpu.sync_copy(meta_hbm.at[wid], meta_v)
        mv = meta_v[...]

        def ext(j):
            return mv[j]

        myrow0 = sid * TROWS
        for ck in range(CPC):
            r0 = (cid * CPC + ck) * CROWS
            s = ext(ck)
            e = ext(ck + 1)
            s8 = (s // 8) * 8
            nt = (e - s8 + K - 1) // K

            for zb in range(TROWS // ZROWS):
                pltpu.sync_copy(
                    zbuf, acc_sh.at[pl.ds(myrow0 + zb * ZROWS, ZROWS)])
            plsc.subcore_barrier()

            def micro(i, _):
                off = s8 + (sid + i * NSUB) * K
                pltpu.sync_copy(bi_hbm.at[pl.ds(off, K)], bi_v)
                pltpu.sync_copy(fi_hbm.at[pl.ds(off, K)], fi_v)
                pltpu.sync_copy(fidx_hbm.at[pl.ds(off, K)], fidx_v)
                pltpu.sync_copy(val_hbm.at[pl.ds(off, K)], vraw_v)
                for j in range(K // 16):
                    sl = pl.ds(j * 16, 16)
                    lr = bi_v[sl] * FIELD_DIM + fi_v[sl] - r0
                    inr = (lr >= 0) & (lr < CROWS)
                    lrow_v[sl] = jnp.where(inr, lr, 0)
                    val_v[sl] = jnp.where(inr, vraw_v[sl], 0.0)
                pltpu.async_copy(table_hbm.at[fidx_v], rows_v, sem).wait()

                def wrow(j, _):
                    v16 = val_v[pl.ds(j * 16, 16)]
                    for i2 in range(16):
                        sv = v16[i2]
                        r = j * 16 + i2
                        for c in range(EMBED // 16):
                            sl = pl.ds(c * 16, 16)
                            rows_v[r, sl] = rows_v[r, sl] * sv
                    return 0

                lax.fori_loop(0, K // 16, wrow, 0)
                pltpu.sync_copy(rows_v, acc_sh.at[lrow_v], add=True)
                return 0

            nmine = jnp.maximum(0, (nt - sid + NSUB - 1) // NSUB)
            lax.fori_loop(0, nmine, micro, 0)
            plsc.subcore_barrier()

            pltpu.sync_copy(
                acc_sh.at[pl.ds(myrow0, TROWS)],
                out_hbm.at[pl.ds(r0 + myrow0, TROWS)])

    return k


_sc_call = _mesh_kernel()


@jax.jit
def _run(meta, bi, fi, fidx, fv, table):
    return _sc_call(meta, bi, fi, fidx, fv, table)


def kernel(feature_embedding, field_idx, field_sub_idx, feature_idx,
           feature_vals, batch_idx):
    del field_sub_idx
    i32 = jnp.int32
    bi = jnp.concatenate(
        [batch_idx.astype(i32), jnp.full((PAD,), BATCH, i32)])
    fi = jnp.concatenate([field_idx.astype(i32), jnp.zeros((PAD,), i32)])
    fx = jnp.concatenate([feature_idx.astype(i32), jnp.zeros((PAD,), i32)])
    fv = jnp.concatenate([feature_vals, jnp.zeros((PAD,), jnp.float32)])
    bounds = jnp.searchsorted(
        batch_idx,
        jnp.arange(0, BATCH + 1, CHUNK_B, dtype=i32)).astype(i32)
    c_of_w = jnp.arange(32, dtype=i32) % NCORE
    cols = CPC * c_of_w[:, None] + jnp.arange(16, dtype=i32)[None, :]
    meta = bounds[jnp.minimum(cols, NCHUNK)]
    return _run(meta, bi, fi, fx, fv, feature_embedding)


# speedup vs baseline: 2.0841x; 1.1294x over previous
import functools

import jax
import jax.numpy as jnp
from jax import lax
from jax.experimental import pallas as pl
from jax.experimental.pallas import tpu as pltpu
from jax.experimental.pallas import tpu_sc as plsc

BATCH = 4096
FIELD_DIM = 26
VOCAB = 100000
EMBED = 64
NNZ = BATCH * FIELD_DIM

NCORE = 2
NSUB = 16
CPC = 2
NCHUNK = NCORE * CPC
CHUNK_B = BATCH // NCHUNK
CROWS = CHUNK_B * FIELD_DIM
TROWS = CROWS // NSUB
K = 128
ZROWS = 104
NPAD = NNZ + 17 * K
NB = NPAD // K


def _mesh_kernel():
    mesh = plsc.VectorSubcoreMesh(core_axis_name="c", subcore_axis_name="s")

    @functools.partial(
        pl.kernel,
        mesh=mesh,
        out_type=jax.ShapeDtypeStruct((NNZ, EMBED), jnp.float32),
        compiler_params=pltpu.CompilerParams(use_tc_tiling_on_sc=False),
        scratch_types=[
            pltpu.VMEM((16,), jnp.int32),
            pltpu.VMEM((3, K), jnp.int32),
            pltpu.VMEM((K,), jnp.float32),
            pltpu.VMEM((K,), jnp.int32),
            pltpu.VMEM((K, EMBED), jnp.float32),
            pltpu.VMEM((ZROWS, EMBED), jnp.float32),
            pltpu.VMEM_SHARED((CROWS + 8, EMBED), jnp.float32),
            pltpu.SemaphoreType.DMA,
        ],
    )
    def k(meta_hbm, pk_hbm, val_hbm, table_hbm, out_hbm,
          meta_v, pk_v, val_v, lrow_v, rows_v, zbuf, acc_sh, sem):
        cid = lax.axis_index("c")
        sid = lax.axis_index("s")
        wid = sid * NCORE + cid
        z16 = jnp.zeros((16,), jnp.float32)

        def zrow(i, _):
            for c in range(EMBED // 16):
                zbuf[i, pl.ds(c * 16, 16)] = z16
            return 0

        lax.fori_loop(0, ZROWS, zrow, 0)

        pltpu.sync_copy(meta_hbm.at[wid], meta_v)
        mv = meta_v[...]

        myrow0 = sid * TROWS
        for ck in range(CPC):
            r0 = (cid * CPC + ck) * CROWS
            s = mv[ck]
            e = mv[ck + 1]
            b0 = s // K
            nt = (e - b0 * K + K - 1) // K

            for zb in range(TROWS // ZROWS):
                pltpu.sync_copy(
                    zbuf, acc_sh.at[pl.ds(myrow0 + zb * ZROWS, ZROWS)])
            plsc.subcore_barrier()

            def micro(i, _):
                blk = b0 + sid + i * NSUB
                pltpu.sync_copy(pk_hbm.at[blk], pk_v)
                gcp = pltpu.async_copy(table_hbm.at[pk_v.at[2]], rows_v, sem)
                pltpu.sync_copy(val_hbm.at[blk], val_v)
                for j in range(K // 16):
                    sl = pl.ds(j * 16, 16)
                    lr = pk_v[0, sl] * FIELD_DIM + pk_v[1, sl] - r0
                    inr = (lr >= 0) & (lr < CROWS)
                    lrow_v[sl] = jnp.where(inr, lr, CROWS)
                gcp.wait()

                def wrow(j, _):
                    v16 = val_v[pl.ds(j * 16, 16)]
                    for i2 in range(16):
                        sv = v16[i2]
                        r = j * 16 + i2
                        for c in range(EMBED // 16):
                            sl = pl.ds(c * 16, 16)
                            rows_v[r, sl] = rows_v[r, sl] * sv
                    return 0

                lax.fori_loop(0, K // 16, wrow, 0)
                pltpu.sync_copy(rows_v, acc_sh.at[lrow_v], add=True)
                return 0

            nmine = jnp.maximum(0, (nt - sid + NSUB - 1) // NSUB)
            lax.fori_loop(0, nmine, micro, 0)
            plsc.subcore_barrier()

            pltpu.sync_copy(
                acc_sh.at[pl.ds(myrow0, TROWS)],
                out_hbm.at[pl.ds(r0 + myrow0, TROWS)])

    return k


_sc_call = _mesh_kernel()


@jax.jit
def _run(meta, packed, vals, table):
    return _sc_call(meta, packed, vals, table)


def kernel(feature_embedding, field_idx, field_sub_idx, feature_idx,
           feature_vals, batch_idx):
    del field_sub_idx
    i32 = jnp.int32
    pad = NPAD - NNZ
    bi = jnp.concatenate([batch_idx.astype(i32), jnp.full((pad,), BATCH, i32)])
    fi = jnp.concatenate([field_idx.astype(i32), jnp.zeros((pad,), i32)])
    fx = jnp.concatenate([feature_idx.astype(i32), jnp.zeros((pad,), i32)])
    fv = jnp.concatenate(
        [feature_vals, jnp.zeros((pad,), jnp.float32)]).reshape(NB, K)
    packed = jnp.stack([bi, fi, fx]).reshape(3, NB, K).transpose(1, 0, 2)
    bounds = jnp.searchsorted(
        batch_idx,
        jnp.arange(0, BATCH + 1, CHUNK_B, dtype=i32)).astype(i32)
    c_of_w = jnp.arange(32, dtype=i32) % NCORE
    cols = CPC * c_of_w[:, None] + jnp.arange(16, dtype=i32)[None, :]
    meta = bounds[jnp.minimum(cols, NCHUNK)]
    return _run(meta, packed, fv, feature_embedding)


# speedup vs baseline: 2.3045x; 1.1058x over previous
import functools

import jax
import jax.numpy as jnp
from jax import lax
from jax.experimental import pallas as pl
from jax.experimental.pallas import tpu as pltpu
from jax.experimental.pallas import tpu_sc as plsc

BATCH = 4096
FIELD_DIM = 26
VOCAB = 100000
EMBED = 64
NNZ = BATCH * FIELD_DIM

NCORE = 2
NSUB = 16
CPC = 2
NCHUNK = NCORE * CPC
CHUNK_B = BATCH // NCHUNK
CROWS = CHUNK_B * FIELD_DIM
TROWS = CROWS // NSUB
K = 128
ZROWS = 104
NPAD = NNZ + 17 * K
NB = NPAD // K


def _mesh_kernel():
    mesh = plsc.VectorSubcoreMesh(core_axis_name="c", subcore_axis_name="s")

    @functools.partial(
        pl.kernel,
        mesh=mesh,
        out_type=jax.ShapeDtypeStruct((NNZ, EMBED), jnp.float32),
        compiler_params=pltpu.CompilerParams(use_tc_tiling_on_sc=False),
        scratch_types=[
            pltpu.VMEM((16,), jnp.int32),
            pltpu.VMEM((3, K), jnp.int32),
            pltpu.VMEM((3, K), jnp.int32),
            pltpu.VMEM((K,), jnp.float32),
            pltpu.VMEM((K,), jnp.float32),
            pltpu.VMEM((K,), jnp.int32),
            pltpu.VMEM((K, EMBED), jnp.float32),
            pltpu.VMEM((K, EMBED), jnp.float32),
            pltpu.VMEM((ZROWS, EMBED), jnp.float32),
            pltpu.VMEM_SHARED((CROWS + 8, EMBED), jnp.float32),
            pltpu.SemaphoreType.DMA,
            pltpu.SemaphoreType.DMA,
            pltpu.SemaphoreType.DMA,
            pltpu.SemaphoreType.DMA,
            pltpu.SemaphoreType.DMA,
            pltpu.SemaphoreType.DMA,
            pltpu.SemaphoreType.DMA,
        ],
    )
    def k(meta_hbm, pk_hbm, val_hbm, table_hbm, out_hbm,
          meta_v, pk_v0, pk_v1, val_v0, val_v1, lrow_v, rows_v0, rows_v1,
          zbuf, acc_sh, sem_meta, sem_pk0, sem_pk1, sem_val0, sem_val1,
          sem_g0, sem_g1):
        cid = lax.axis_index("c")
        sid = lax.axis_index("s")
        wid = sid * NCORE + cid
        z16 = jnp.zeros((16,), jnp.float32)
        pk_v = (pk_v0, pk_v1)
        val_v = (val_v0, val_v1)
        rows_v = (rows_v0, rows_v1)
        sem_pk = (sem_pk0, sem_pk1)
        sem_val = (sem_val0, sem_val1)
        sem_g = (sem_g0, sem_g1)

        def zrow(i, _):
            for c in range(EMBED // 16):
                zbuf[i, pl.ds(c * 16, 16)] = z16
            return 0

        lax.fori_loop(0, ZROWS, zrow, 0)

        pltpu.sync_copy(meta_hbm.at[wid], meta_v)
        mv = meta_v[...]

        myrow0 = sid * TROWS
        for ck in range(CPC):
            r0 = (cid * CPC + ck) * CROWS
            s = mv[ck]
            e = mv[ck + 1]
            b0 = s // K
            nt = (e - b0 * K + K - 1) // K
            n = jnp.maximum(0, (nt - sid + NSUB - 1) // NSUB)

            for zb in range(TROWS // ZROWS):
                pltpu.sync_copy(
                    zbuf, acc_sh.at[pl.ds(myrow0 + zb * ZROWS, ZROWS)])
            plsc.subcore_barrier()

            def blk(i):
                return b0 + sid + i * NSUB

            def start_in(i, sl):
                pltpu.async_copy(pk_hbm.at[blk(i)], pk_v[sl], sem_pk[sl])
                pltpu.async_copy(val_hbm.at[blk(i)], val_v[sl], sem_val[sl])

            def wait_pk(sl):
                pltpu.make_async_copy(
                    pk_hbm.at[0], pk_v[sl], sem_pk[sl]).wait()

            def wait_val(sl):
                pltpu.make_async_copy(
                    val_hbm.at[0], val_v[sl], sem_val[sl]).wait()

            def start_gather(sl):
                pltpu.async_copy(
                    table_hbm.at[pk_v[sl].at[2]], rows_v[sl], sem_g[sl])

            def wait_gather(sl):
                pltpu.make_async_copy(
                    table_hbm.at[pl.ds(0, K)], rows_v[sl], sem_g[sl]).wait()

            @pl.when(n > 0)
            def _():
                start_in(0, 0)
                wait_pk(0)
                start_gather(0)

            @pl.when(n > 1)
            def _():
                start_in(1, 1)

            def body(i, sl):
                so = 1 - sl
                for j in range(K // 16):
                    s16 = pl.ds(j * 16, 16)
                    lr = (pk_v[sl][0, s16] * FIELD_DIM
                          + pk_v[sl][1, s16] - r0)
                    inr = (lr >= 0) & (lr < CROWS)
                    lrow_v[s16] = jnp.where(inr, lr, CROWS)
                wait_gather(sl)

                @pl.when(i + 1 < n)
                def _():
                    wait_pk(so)
                    start_gather(so)

                wait_val(sl)

                def wrow(j, _):
                    v16 = val_v[sl][pl.ds(j * 16, 16)]
                    for i2 in range(16):
                        sv = v16[i2]
                        r = j * 16 + i2
                        for c in range(EMBED // 16):
                            s16 = pl.ds(c * 16, 16)
                            rows_v[sl][r, s16] = rows_v[sl][r, s16] * sv
                    return 0

                lax.fori_loop(0, K // 16, wrow, 0)
                pltpu.sync_copy(rows_v[sl], acc_sh.at[lrow_v], add=True)

                @pl.when(i + 2 < n)
                def _():
                    start_in(i + 2, sl)

            def pair(t, _):
                for sl in range(2):
                    i = 2 * t + sl

                    @pl.when(i < n)
                    def _():
                        body(i, sl)

                return 0

            lax.fori_loop(0, (n + 1) // 2, pair, 0)
            plsc.subcore_barrier()

            pltpu.sync_copy(
                acc_sh.at[pl.ds(myrow0, TROWS)],
                out_hbm.at[pl.ds(r0 + myrow0, TROWS)])

    return k


_sc_call = _mesh_kernel()


@jax.jit
def _run(meta, packed, vals, table):
    return _sc_call(meta, packed, vals, table)


def kernel(feature_embedding, field_idx, field_sub_idx, feature_idx,
           feature_vals, batch_idx):
    del field_sub_idx
    i32 = jnp.int32
    pad = NPAD - NNZ
    bi = jnp.concatenate([batch_idx.astype(i32), jnp.full((pad,), BATCH, i32)])
    fi = jnp.concatenate([field_idx.astype(i32), jnp.zeros((pad,), i32)])
    fx = jnp.concatenate([feature_idx.astype(i32), jnp.zeros((pad,), i32)])
    fv = jnp.concatenate(
        [feature_vals, jnp.zeros((pad,), jnp.float32)]).reshape(NB, K)
    packed = jnp.stack([bi, fi, fx]).reshape(3, NB, K).transpose(1, 0, 2)
    bounds = jnp.searchsorted(
        batch_idx,
        jnp.arange(0, BATCH + 1, CHUNK_B, dtype=i32)).astype(i32)
    c_of_w = jnp.arange(32, dtype=i32) % NCORE
    cols = CPC * c_of_w[:, None] + jnp.arange(16, dtype=i32)[None, :]
    meta = bounds[jnp.minimum(cols, NCHUNK)]
    return _run(meta, packed, fv, feature_embedding)
